# baseline (speedup 1.0000x reference)
import functools

import jax
import jax.numpy as jnp
from jax import lax
from jax.experimental import pallas as pl
from jax.experimental.pallas import tpu as pltpu
from jax.experimental.pallas import tpu_sc as plsc

N = 10000
NP = 10240
E = 320000
D_FEAT = 128
HIDDEN = 64
C = 40
KPROP = 10
LAM = 0.5

NC = 2
NS = 16
NW = NC * NS
G = 8
CPG = C // G
ESPLIT = NW // G
EPT = E // ESPLIT
CH = 2000
NN = NP // NW

_MESH = functools.partial(
    plsc.VectorSubcoreMesh, core_axis_name="c", subcore_axis_name="s")
_SC_PARAMS = pltpu.CompilerParams(
    needs_layout_passes=False, use_tc_tiling_on_sc=False)


def _wid():
    return lax.axis_index("s") * NC + lax.axis_index("c")


@functools.partial(
    pl.kernel,
    mesh=_MESH(),
    compiler_params=_SC_PARAMS,
    out_type=jax.ShapeDtypeStruct((NW, 1, N), jnp.float32),
    scratch_types=[
        pltpu.VMEM((N,), jnp.float32),
        pltpu.VMEM((E // NW,), jnp.int32),
        pltpu.SemaphoreType.DMA,
    ],
)
def _sc_deg(dst_hbm, out_hbm, acc, ibuf, sem):
    wid = _wid()
    zero16 = jnp.zeros((16,), jnp.float32)
    ones16 = jnp.ones((16,), jnp.float32)
    ept = E // NW

    cp = pltpu.make_async_copy(dst_hbm.at[pl.ds(wid * ept, ept)], ibuf, sem)
    cp.start()

    @plsc.parallel_loop(0, N // 16, unroll=8)
    def zbody(i):
        acc[pl.ds(i * 16, 16)] = zero16

    cp.wait()

    @plsc.parallel_loop(0, ept // 16, unroll=8)
    def jbody(j):
        idx = ibuf[pl.ds(j * 16, 16)]
        plsc.addupdate_scatter(acc, [idx], ones16)

    pltpu.sync_copy(acc, out_hbm.at[wid, 0])


@functools.partial(
    pl.kernel,
    mesh=_MESH(),
    compiler_params=_SC_PARAMS,
    out_type=jax.ShapeDtypeStruct((NW, CPG, NP), jnp.float32),
    scratch_types=[
        pltpu.VMEM((CPG, NP), jnp.float32),
        pltpu.VMEM((CPG, NP), jnp.float32),
        pltpu.VMEM((2, CH), jnp.int32),
        pltpu.VMEM((2, CH), jnp.int32),
        pltpu.SemaphoreType.DMA,
        pltpu.SemaphoreType.DMA,
    ],
)
def _sc_spmm(src_hbm, dst_hbm, xk_hbm, out_hbm,
             xs, acc, sbuf, dbuf, sem0, sem1):
    wid = _wid()
    e = wid // G
    g = wid % G
    sems = (sem0, sem1)
    nch = EPT // CH

    def copies(k, slot):
        base = e * EPT + k * CH
        sem = sems[slot]
        return (
            pltpu.make_async_copy(src_hbm.at[pl.ds(base, CH)],
                                  sbuf.at[slot], sem),
            pltpu.make_async_copy(dst_hbm.at[pl.ds(base, CH)],
                                  dbuf.at[slot], sem),
        )

    def start_chunk(k, slot):
        for cp in copies(k, slot):
            cp.start()

    def wait_chunk(k, slot):
        for cp in copies(k, slot):
            cp.wait()

    start_chunk(0, 0)
    pltpu.sync_copy(xk_hbm.at[g], xs)

    zero16 = jnp.zeros((16,), jnp.float32)
    for c in range(CPG):
        @plsc.parallel_loop(0, NP // 16, unroll=8)
        def zbody(i):
            acc[c, pl.ds(i * 16, 16)] = zero16

    cvecs = [jnp.full((16,), c, jnp.int32) for c in range(CPG)]

    def process(k, slot):
        @plsc.parallel_loop(0, CH // 16, unroll=8)
        def jbody(j):
            s = sbuf[slot, pl.ds(j * 16, 16)]
            d = dbuf[slot, pl.ds(j * 16, 16)]
            for c in range(CPG):
                v = plsc.load_gather(xs, [cvecs[c], s])
                plsc.addupdate_scatter(acc, [cvecs[c], d], v)

    def cbody(k2, carry):
        k = k2 * 2

        start_chunk(k + 1, 1)
        wait_chunk(k, 0)
        process(k, 0)

        @pl.when(k + 2 < nch)
        def _():
            start_chunk(k + 2, 0)

        wait_chunk(k + 1, 1)
        process(k + 1, 1)
        return carry

    lax.fori_loop(0, nch // 2, cbody, 0)
    pltpu.sync_copy(acc, out_hbm.at[wid])


@functools.partial(
    pl.kernel,
    mesh=_MESH(),
    compiler_params=_SC_PARAMS,
    out_type=jax.ShapeDtypeStruct((G, CPG, NP), jnp.float32),
    scratch_types=[
        pltpu.VMEM((NW, CPG, NN), jnp.float32),
        pltpu.VMEM((G, CPG, NN), jnp.float32),
        pltpu.VMEM((G, CPG, NN), jnp.float32),
        pltpu.VMEM((G, CPG, NN), jnp.float32),
        pltpu.VMEM((G, CPG, NN), jnp.float32),
        pltpu.VMEM((NN,), jnp.float32),
        pltpu.SemaphoreType.DMA,
    ],
)
def _sc_prox(parts_hbm, yk_hbm, hh_hbm, dinv_hbm, out_hbm,
             pbuf, ykb, hhb, zb, ob, dvb, sem):
    wid = _wid()
    n0 = wid * NN

    cps = (
        pltpu.make_async_copy(parts_hbm.at[:, :, pl.ds(n0, NN)], pbuf, sem),
        pltpu.make_async_copy(yk_hbm.at[:, :, pl.ds(n0, NN)], ykb, sem),
        pltpu.make_async_copy(hh_hbm.at[:, :, pl.ds(n0, NN)], hhb, sem),
        pltpu.make_async_copy(dinv_hbm.at[pl.ds(n0, NN)], dvb, sem),
    )
    for cp in cps:
        cp.start()
    for cp in cps:
        cp.wait()

    magic = jnp.full((16,), 0x5F3759DF, jnp.int32)

    @plsc.parallel_loop(0, NN // 16, unroll=2)
    def jbody(j):
        ds = pl.ds(j * 16, 16)
        dv = dvb[ds]
        rn2 = jnp.zeros((16,), jnp.float32)
        for g in range(G):
            for c in range(CPG):
                q = (pbuf[0 * G + g, c, ds] + pbuf[1 * G + g, c, ds]
                     + pbuf[2 * G + g, c, ds] + pbuf[3 * G + g, c, ds])
                z = dv * (q + ykb[g, c, ds]) - hhb[g, c, ds]
                zb[g, c, ds] = z
                rn2 = rn2 + z * z
        i = plsc.bitcast(rn2, jnp.int32)
        i = magic - lax.shift_right_logical(i, 1)
        y = plsc.bitcast(i, jnp.float32)
        for _ in range(3):
            y = y * (1.5 - 0.5 * rn2 * y * y)
        score = jnp.maximum(1.0 - LAM * y, 0.0)
        for g in range(G):
            for c in range(CPG):
                ob[g, c, ds] = dv * (hhb[g, c, ds] + score * zb[g, c, ds])

    pltpu.sync_copy(ob, out_hbm.at[:, :, pl.ds(n0, NN)])


def _tc_mlp(x, W1, b1, W2, b2):
    mb = 2000

    def body(x_ref, w1_ref, b1_ref, w2_ref, b2_ref, out_ref):
        hmid = jnp.dot(x_ref[...], w1_ref[...],
                       preferred_element_type=jnp.float32) + b1_ref[...]
        hmid = jnp.maximum(hmid, 0.0)
        out_ref[...] = jnp.dot(hmid, w2_ref[...],
                               preferred_element_type=jnp.float32) + b2_ref[...]

    return pl.pallas_call(
        body,
        grid=(N // mb,),
        in_specs=[
            pl.BlockSpec((mb, D_FEAT), lambda i: (i, 0)),
            pl.BlockSpec((D_FEAT, HIDDEN), lambda i: (0, 0)),
            pl.BlockSpec((1, HIDDEN), lambda i: (0, 0)),
            pl.BlockSpec((HIDDEN, C), lambda i: (0, 0)),
            pl.BlockSpec((1, C), lambda i: (0, 0)),
        ],
        out_specs=pl.BlockSpec((mb, C), lambda i: (i, 0)),
        out_shape=jax.ShapeDtypeStruct((N, C), jnp.float32),
    )(x, W1, b1, W2, b2)


def _tc_dinv(partials):
    def body(p_ref, dinv_ref, dsqrt_ref):
        deg = jnp.sum(p_ref[...], axis=0, keepdims=True) + 1.0
        dinv = lax.rsqrt(deg)
        dinv_ref[...] = dinv
        dsqrt_ref[...] = deg * dinv

    return pl.pallas_call(
        body,
        out_shape=[jax.ShapeDtypeStruct((1, 1, N), jnp.float32),
                   jax.ShapeDtypeStruct((1, 1, N), jnp.float32)],
    )(partials)


def _tc_scale(hh3, dinvp):
    def body(hh_ref, dv_ref, out_ref):
        out_ref[...] = hh_ref[...] * dv_ref[...][None, None, :]

    return pl.pallas_call(
        body,
        out_shape=jax.ShapeDtypeStruct((G, CPG, NP), jnp.float32),
    )(hh3, dinvp)


def _tc_logsoftmax(yk3, dsqrtp):
    def body(yk_ref, dsq_ref, out_ref):
        z = yk_ref[...] * dsq_ref[...][None, None, :]
        m = jnp.max(jnp.max(z, axis=1, keepdims=True), axis=0, keepdims=True)
        ez = jnp.exp(z - m)
        lse = jnp.log(jnp.sum(jnp.sum(ez, axis=1, keepdims=True),
                              axis=0, keepdims=True))
        out_ref[...] = z - m - lse

    return pl.pallas_call(
        body,
        out_shape=jax.ShapeDtypeStruct((G, CPG, NP), jnp.float32),
    )(yk3, dsqrtp)


def kernel(x, edge_index, W1, b1, W2, b2):
    src = edge_index[0]
    dst = edge_index[1]

    h = _tc_mlp(x, W1, b1.reshape(1, HIDDEN), W2, b2.reshape(1, C))
    hh3 = jnp.pad(h.T.reshape(G, CPG, N), ((0, 0), (0, 0), (0, NP - N)))

    degp = _sc_deg(dst)
    dinv, dsqrt = _tc_dinv(degp)
    dinvp = jnp.pad(dinv.reshape(N), (0, NP - N))
    dsqrtp = jnp.pad(dsqrt.reshape(N), (0, NP - N))

    yk3 = _tc_scale(hh3, dinvp)
    for _ in range(KPROP):
        parts = _sc_spmm(src, dst, yk3)
        yk3 = _sc_prox(parts, yk3, hh3, dinvp)

    out3 = _tc_logsoftmax(yk3, dsqrtp)
    return out3[:, :, :N].reshape(C, N).T


# speedup vs baseline: 1.0617x; 1.0617x over previous
import functools

import jax
import jax.numpy as jnp
from jax import lax
from jax.experimental import pallas as pl
from jax.experimental.pallas import tpu as pltpu
from jax.experimental.pallas import tpu_sc as plsc

N = 10000
NP = 10240
E = 320000
D_FEAT = 128
HIDDEN = 64
C = 40
KPROP = 10
LAM = 0.5

NC = 2
NS = 16
NW = NC * NS
G = 8
CPG = C // G
ESPLIT = NW // G
EPT = E // ESPLIT
CH = 4000
NN = NP // NW

_MESH = functools.partial(
    plsc.VectorSubcoreMesh, core_axis_name="c", subcore_axis_name="s")
_SC_PARAMS = pltpu.CompilerParams(
    needs_layout_passes=False, use_tc_tiling_on_sc=False)


def _wid():
    return lax.axis_index("s") * NC + lax.axis_index("c")


@functools.partial(
    pl.kernel,
    mesh=_MESH(),
    compiler_params=_SC_PARAMS,
    out_type=jax.ShapeDtypeStruct((NW, 1, N), jnp.float32),
    scratch_types=[
        pltpu.VMEM((N,), jnp.float32),
        pltpu.VMEM((E // NW,), jnp.int32),
        pltpu.SemaphoreType.DMA,
    ],
)
def _sc_deg(dst_hbm, out_hbm, acc, ibuf, sem):
    wid = _wid()
    zero16 = jnp.zeros((16,), jnp.float32)
    ones16 = jnp.ones((16,), jnp.float32)
    ept = E // NW

    cp = pltpu.make_async_copy(dst_hbm.at[pl.ds(wid * ept, ept)], ibuf, sem)
    cp.start()

    @plsc.parallel_loop(0, N // 16, unroll=8)
    def zbody(i):
        acc[pl.ds(i * 16, 16)] = zero16

    cp.wait()

    @plsc.parallel_loop(0, ept // 16, unroll=8)
    def jbody(j):
        idx = ibuf[pl.ds(j * 16, 16)]
        plsc.addupdate_scatter(acc, [idx], ones16)

    pltpu.sync_copy(acc, out_hbm.at[wid, 0])


@functools.partial(
    pl.kernel,
    mesh=_MESH(),
    compiler_params=_SC_PARAMS,
    out_type=jax.ShapeDtypeStruct((NW, CPG, NP), jnp.float32),
    scratch_types=[
        pltpu.VMEM((CPG, NP), jnp.float32),
        pltpu.VMEM((CPG, NP), jnp.float32),
        pltpu.VMEM((2, CH), jnp.int32),
        pltpu.VMEM((2, CH), jnp.int32),
        pltpu.SemaphoreType.DMA,
        pltpu.SemaphoreType.DMA,
    ],
)
def _sc_spmm(src_hbm, dst_hbm, xk_hbm, out_hbm,
             xs, acc, sbuf, dbuf, sem0, sem1):
    wid = _wid()
    e = wid // G
    g = wid % G
    sems = (sem0, sem1)
    nch = EPT // CH

    def copies(k, slot):
        base = e * EPT + k * CH
        sem = sems[slot]
        return (
            pltpu.make_async_copy(src_hbm.at[pl.ds(base, CH)],
                                  sbuf.at[slot], sem),
            pltpu.make_async_copy(dst_hbm.at[pl.ds(base, CH)],
                                  dbuf.at[slot], sem),
        )

    def start_chunk(k, slot):
        for cp in copies(k, slot):
            cp.start()

    def wait_chunk(k, slot):
        for cp in copies(k, slot):
            cp.wait()

    start_chunk(0, 0)
    pltpu.sync_copy(xk_hbm.at[g], xs)

    zero16 = jnp.zeros((16,), jnp.float32)
    for c in range(CPG):
        @plsc.parallel_loop(0, NP // 16, unroll=8)
        def zbody(i):
            acc[c, pl.ds(i * 16, 16)] = zero16

    cvecs = [jnp.full((16,), c, jnp.int32) for c in range(CPG)]

    def process(k, slot):
        @plsc.parallel_loop(0, CH // 16, unroll=5)
        def jbody(j):
            s = sbuf[slot, pl.ds(j * 16, 16)]
            d = dbuf[slot, pl.ds(j * 16, 16)]
            for c in range(CPG):
                v = plsc.load_gather(xs, [cvecs[c], s])
                plsc.addupdate_scatter(acc, [cvecs[c], d], v)

    def cbody(k2, carry):
        k = k2 * 2

        start_chunk(k + 1, 1)
        wait_chunk(k, 0)
        process(k, 0)

        @pl.when(k + 2 < nch)
        def _():
            start_chunk(k + 2, 0)

        wait_chunk(k + 1, 1)
        process(k + 1, 1)
        return carry

    lax.fori_loop(0, nch // 2, cbody, 0)
    pltpu.sync_copy(acc, out_hbm.at[wid])


@functools.partial(
    pl.kernel,
    mesh=_MESH(),
    compiler_params=_SC_PARAMS,
    out_type=jax.ShapeDtypeStruct((G, CPG, NP), jnp.float32),
    scratch_types=[
        pltpu.VMEM((NW, CPG, NN), jnp.float32),
        pltpu.VMEM((G, CPG, NN), jnp.float32),
        pltpu.VMEM((G, CPG, NN), jnp.float32),
        pltpu.VMEM((G, CPG, NN), jnp.float32),
        pltpu.VMEM((G, CPG, NN), jnp.float32),
        pltpu.VMEM((NN,), jnp.float32),
        pltpu.SemaphoreType.DMA,
    ],
)
def _sc_prox(parts_hbm, yk_hbm, hh_hbm, dinv_hbm, out_hbm,
             pbuf, ykb, hhb, zb, ob, dvb, sem):
    wid = _wid()
    n0 = wid * NN

    cps = (
        pltpu.make_async_copy(parts_hbm.at[:, :, pl.ds(n0, NN)], pbuf, sem),
        pltpu.make_async_copy(yk_hbm.at[:, :, pl.ds(n0, NN)], ykb, sem),
        pltpu.make_async_copy(hh_hbm.at[:, :, pl.ds(n0, NN)], hhb, sem),
        pltpu.make_async_copy(dinv_hbm.at[pl.ds(n0, NN)], dvb, sem),
    )
    for cp in cps:
        cp.start()
    for cp in cps:
        cp.wait()

    magic = jnp.full((16,), 0x5F3759DF, jnp.int32)

    @plsc.parallel_loop(0, NN // 16, unroll=1)
    def jbody(j):
        ds = pl.ds(j * 16, 16)
        dv = dvb[ds]
        rn2 = jnp.zeros((16,), jnp.float32)
        for g in range(G):
            for c in range(CPG):
                q = (pbuf[0 * G + g, c, ds] + pbuf[1 * G + g, c, ds]
                     + pbuf[2 * G + g, c, ds] + pbuf[3 * G + g, c, ds])
                z = dv * (q + ykb[g, c, ds]) - hhb[g, c, ds]
                zb[g, c, ds] = z
                rn2 = rn2 + z * z
        i = plsc.bitcast(rn2, jnp.int32)
        i = magic - lax.shift_right_logical(i, 1)
        y = plsc.bitcast(i, jnp.float32)
        for _ in range(3):
            y = y * (1.5 - 0.5 * rn2 * y * y)
        score = jnp.maximum(1.0 - LAM * y, 0.0)
        for g in range(G):
            for c in range(CPG):
                ob[g, c, ds] = dv * (hhb[g, c, ds] + score * zb[g, c, ds])

    pltpu.sync_copy(ob, out_hbm.at[:, :, pl.ds(n0, NN)])


def _tc_mlp(x, W1, b1, W2, b2):
    mb = 2000

    def body(x_ref, w1_ref, b1_ref, w2_ref, b2_ref, out_ref):
        hmid = jnp.dot(x_ref[...], w1_ref[...],
                       preferred_element_type=jnp.float32) + b1_ref[...]
        hmid = jnp.maximum(hmid, 0.0)
        out_ref[...] = jnp.dot(hmid, w2_ref[...],
                               preferred_element_type=jnp.float32) + b2_ref[...]

    return pl.pallas_call(
        body,
        grid=(N // mb,),
        in_specs=[
            pl.BlockSpec((mb, D_FEAT), lambda i: (i, 0)),
            pl.BlockSpec((D_FEAT, HIDDEN), lambda i: (0, 0)),
            pl.BlockSpec((1, HIDDEN), lambda i: (0, 0)),
            pl.BlockSpec((HIDDEN, C), lambda i: (0, 0)),
            pl.BlockSpec((1, C), lambda i: (0, 0)),
        ],
        out_specs=pl.BlockSpec((mb, C), lambda i: (i, 0)),
        out_shape=jax.ShapeDtypeStruct((N, C), jnp.float32),
    )(x, W1, b1, W2, b2)


def _tc_dinv(partials):
    def body(p_ref, dinv_ref, dsqrt_ref):
        deg = jnp.sum(p_ref[...], axis=0, keepdims=True) + 1.0
        dinv = lax.rsqrt(deg)
        dinv_ref[...] = dinv
        dsqrt_ref[...] = deg * dinv

    return pl.pallas_call(
        body,
        out_shape=[jax.ShapeDtypeStruct((1, 1, N), jnp.float32),
                   jax.ShapeDtypeStruct((1, 1, N), jnp.float32)],
    )(partials)


def _tc_scale(hh3, dinvp):
    def body(hh_ref, dv_ref, out_ref):
        out_ref[...] = hh_ref[...] * dv_ref[...][None, None, :]

    return pl.pallas_call(
        body,
        out_shape=jax.ShapeDtypeStruct((G, CPG, NP), jnp.float32),
    )(hh3, dinvp)


def _tc_logsoftmax(yk3, dsqrtp):
    def body(yk_ref, dsq_ref, out_ref):
        z = yk_ref[...] * dsq_ref[...][None, None, :]
        m = jnp.max(jnp.max(z, axis=1, keepdims=True), axis=0, keepdims=True)
        ez = jnp.exp(z - m)
        lse = jnp.log(jnp.sum(jnp.sum(ez, axis=1, keepdims=True),
                              axis=0, keepdims=True))
        out_ref[...] = z - m - lse

    return pl.pallas_call(
        body,
        out_shape=jax.ShapeDtypeStruct((G, CPG, NP), jnp.float32),
    )(yk3, dsqrtp)


def kernel(x, edge_index, W1, b1, W2, b2):
    src = edge_index[0]
    dst = edge_index[1]

    h = _tc_mlp(x, W1, b1.reshape(1, HIDDEN), W2, b2.reshape(1, C))
    hh3 = jnp.pad(h.T.reshape(G, CPG, N), ((0, 0), (0, 0), (0, NP - N)))

    degp = _sc_deg(dst)
    dinv, dsqrt = _tc_dinv(degp)
    dinvp = jnp.pad(dinv.reshape(N), (0, NP - N))
    dsqrtp = jnp.pad(dsqrt.reshape(N), (0, NP - N))

    yk3 = _tc_scale(hh3, dinvp)
    for _ in range(KPROP):
        parts = _sc_spmm(src, dst, yk3)
        yk3 = _sc_prox(parts, yk3, hh3, dinvp)

    out3 = _tc_logsoftmax(yk3, dsqrtp)
    return out3[:, :, :N].reshape(C, N).T
